# baseline (speedup 1.0000x reference)
import functools

import jax
import jax.numpy as jnp
from jax import lax
from jax.experimental import pallas as pl
from jax.experimental.pallas import tpu as pltpu
from jax.experimental.pallas import tpu_sc as plsc

NUM_EMB = 1000000
D = 64
B = 16384 * 20

_info = plsc.get_sparse_core_info()
NC, NS = _info.num_cores, _info.num_subcores
NW = NC * NS
BPW = B // NW
CHUNK = 128
NCHUNK = BPW // CHUNK

_mesh = plsc.VectorSubcoreMesh(core_axis_name="c", subcore_axis_name="s")


@functools.partial(
    pl.kernel,
    mesh=_mesh,
    out_type=jax.ShapeDtypeStruct((B, D), jnp.float32),
    scratch_types=[
        pltpu.VMEM((BPW,), jnp.int32),
        pltpu.VMEM((CHUNK, D), jnp.float32),
        pltpu.SemaphoreType.DMA,
    ],
    compiler_params=pltpu.CompilerParams(use_tc_tiling_on_sc=False),
)
def _gather(tids_hbm, table_hbm, out_hbm, idx_v, rows_v, sem):
    wid = lax.axis_index("s") * NC + lax.axis_index("c")
    base = wid * BPW
    pltpu.sync_copy(tids_hbm.at[pl.ds(base, BPW)], idx_v)

    def chunk_body(c, carry):
        off = c * CHUNK
        pltpu.async_copy(
            table_hbm.at[idx_v.at[pl.ds(off, CHUNK)]], rows_v, sem
        ).wait()
        pltpu.sync_copy(rows_v, out_hbm.at[pl.ds(base + off, CHUNK)])
        return carry

    lax.fori_loop(0, NCHUNK, chunk_body, 0)


def kernel(token_ids, weight):
    flat = token_ids.reshape(-1).astype(jnp.int32)
    out = _gather(flat, weight)
    return out.reshape(token_ids.shape + (weight.shape[1],))


# speedup vs baseline: 1.0623x; 1.0623x over previous
import functools

import jax
import jax.numpy as jnp
from jax import lax
from jax.experimental import pallas as pl
from jax.experimental.pallas import tpu as pltpu
from jax.experimental.pallas import tpu_sc as plsc

NUM_EMB = 1000000
D = 64
B = 16384 * 20

_info = plsc.get_sparse_core_info()
NC, NS = _info.num_cores, _info.num_subcores
NW = NC * NS
BPW = B // NW
CHUNK = 128
NCHUNK = BPW // CHUNK
NBUF = 8

_mesh = plsc.VectorSubcoreMesh(core_axis_name="c", subcore_axis_name="s")

_scratch = (
    [pltpu.VMEM((BPW,), jnp.int32)]
    + [pltpu.VMEM((CHUNK, D), jnp.float32) for _ in range(NBUF)]
    + [pltpu.SemaphoreType.DMA for _ in range(2 * NBUF)]
)


@functools.partial(
    pl.kernel,
    mesh=_mesh,
    out_type=jax.ShapeDtypeStruct((B, D), jnp.float32),
    scratch_types=_scratch,
    compiler_params=pltpu.CompilerParams(use_tc_tiling_on_sc=False),
)
def _gather(tids_hbm, table_hbm, out_hbm, idx_v, *rest):
    rows = rest[:NBUF]
    sem_g = rest[NBUF : 2 * NBUF]
    sem_w = rest[2 * NBUF :]

    wid = lax.axis_index("s") * NC + lax.axis_index("c")
    base = wid * BPW
    pltpu.sync_copy(tids_hbm.at[pl.ds(base, BPW)], idx_v)

    def fire_gather(c, b):
        pltpu.async_copy(
            table_hbm.at[idx_v.at[pl.ds(c * CHUNK, CHUNK)]], rows[b], sem_g[b]
        )

    def wait_gather(c, b):
        pltpu.make_async_copy(
            table_hbm.at[idx_v.at[pl.ds(c * CHUNK, CHUNK)]], rows[b], sem_g[b]
        ).wait()

    def out_slice(c):
        return out_hbm.at[pl.ds(base + c * CHUNK, CHUNK)]

    for b in range(NBUF):
        fire_gather(b, b)

    def outer(j, carry):
        for b in range(NBUF):
            c = j * NBUF + b
            wait_gather(c, b)
            wb = pltpu.async_copy(rows[b], out_slice(c), sem_w[b])
            wb.wait()
            fire_gather(c + NBUF, b)
        return carry

    lax.fori_loop(0, (NCHUNK - NBUF) // NBUF, outer, 0)

    for b in range(NBUF):
        c = NCHUNK - NBUF + b
        wait_gather(c, b)
        pltpu.async_copy(rows[b], out_slice(c), sem_w[b]).wait()


def kernel(token_ids, weight):
    flat = token_ids.reshape(-1).astype(jnp.int32)
    out = _gather(flat, weight)
    return out.reshape(token_ids.shape + (weight.shape[1],))


# speedup vs baseline: 1.1036x; 1.0389x over previous
import functools

import jax
import jax.numpy as jnp
from jax import lax
from jax.experimental import pallas as pl
from jax.experimental.pallas import tpu as pltpu
from jax.experimental.pallas import tpu_sc as plsc

NUM_EMB = 1000000
D = 64
B = 16384 * 20

_info = plsc.get_sparse_core_info()
NC, NS = _info.num_cores, _info.num_subcores
NW = NC * NS
BPW = B // NW
CHUNK = 128
NCHUNK = BPW // CHUNK
NBUF = 8

_mesh = plsc.VectorSubcoreMesh(core_axis_name="c", subcore_axis_name="s")

_scratch = (
    [pltpu.VMEM((BPW,), jnp.int32)]
    + [pltpu.VMEM((CHUNK, D), jnp.float32) for _ in range(NBUF)]
    + [pltpu.SemaphoreType.DMA for _ in range(2 * NBUF)]
)


@functools.partial(
    pl.kernel,
    mesh=_mesh,
    out_type=jax.ShapeDtypeStruct((20, 16384, D), jnp.float32),
    scratch_types=_scratch,
    compiler_params=pltpu.CompilerParams(use_tc_tiling_on_sc=False),
)
def _gather(tids_hbm, table_hbm, out_hbm, idx_v, *rest):
    rows = rest[:NBUF]
    sem_g = rest[NBUF : 2 * NBUF]
    sem_w = rest[2 * NBUF :]

    wid = lax.axis_index("s") * NC + lax.axis_index("c")
    base = wid * BPW
    pltpu.sync_copy(tids_hbm.at[pl.ds(base, BPW)], idx_v)

    def fire_gather(c, b):
        pltpu.async_copy(
            table_hbm.at[idx_v.at[pl.ds(c * CHUNK, CHUNK)]], rows[b], sem_g[b]
        )

    def wait_gather(c, b):
        pltpu.make_async_copy(
            table_hbm.at[idx_v.at[pl.ds(c * CHUNK, CHUNK)]], rows[b], sem_g[b]
        ).wait()

    def out_slice(c):
        g = base + c * CHUNK
        return out_hbm.at[g // 16384, pl.ds(g % 16384, CHUNK)]

    for b in range(NBUF):
        fire_gather(b, b)

    def outer(j, carry):
        for b in range(NBUF):
            c = j * NBUF + b
            wait_gather(c, b)
            wb = pltpu.async_copy(rows[b], out_slice(c), sem_w[b])
            wb.wait()
            fire_gather(c + NBUF, b)
        return carry

    lax.fori_loop(0, (NCHUNK - NBUF) // NBUF, outer, 0)

    for b in range(NBUF):
        c = NCHUNK - NBUF + b
        wait_gather(c, b)
        pltpu.async_copy(rows[b], out_slice(c), sem_w[b]).wait()


def kernel(token_ids, weight):
    flat = token_ids.T.reshape(-1).astype(jnp.int32)
    out = _gather(flat, weight)
    return out.transpose(1, 0, 2)
